# baseline (speedup 1.0000x reference)
import functools

import jax
import jax.numpy as jnp
from jax import lax
from jax.experimental import pallas as pl
from jax.experimental.pallas import tpu as pltpu
from jax.experimental.pallas import tpu_sc as plsc

N = 10000
E = 320000
D = 128

NC = 2
NS = 16
NW = NC * NS
B = 125
K = E // (NW * B)
RPT = N // NS
DPT = 640
NDEG = NS * DPT
DW = 16


def _sc_aggregate(x, edge3, zeros_deg, ones_col):
    mesh = plsc.VectorSubcoreMesh(core_axis_name="c", subcore_axis_name="s")

    @functools.partial(
        pl.kernel,
        mesh=mesh,
        out_type=(
            jax.ShapeDtypeStruct((N, D), jnp.float32),
            jax.ShapeDtypeStruct((N, D), jnp.float32),
            jax.ShapeDtypeStruct((NDEG, DW), jnp.float32),
            jax.ShapeDtypeStruct((NDEG, DW), jnp.float32),
        ),
        scratch_types=[
            pltpu.VMEM((4, B), jnp.int32),
            pltpu.VMEM((4, B), jnp.int32),
            pltpu.VMEM((B, D), jnp.float32),
            pltpu.VMEM((B, D), jnp.float32),
            pltpu.VMEM((B, DW), jnp.float32),
            pltpu.VMEM_SHARED((N, D), jnp.float32),
            pltpu.VMEM_SHARED((NDEG, DW), jnp.float32),
            pltpu.SemaphoreType.DMA,
            pltpu.SemaphoreType.DMA,
            pltpu.SemaphoreType.DMA,
            pltpu.SemaphoreType.DMA,
            pltpu.SemaphoreType.DMA,
            pltpu.SemaphoreType.DMA,
            pltpu.SemaphoreType.DMA,
        ],
        compiler_params=pltpu.CompilerParams(use_tc_tiling_on_sc=False),
    )
    def sc_kernel(x_hbm, e_hbm, zd_hbm, ones_hbm,
                  acc0_out, acc1_out, deg0_out, deg1_out,
                  src_v, dst_v, rows_a, rows_b, ones_v,
                  acc_sh, deg_sh, sem_a, sem_b, sem_d,
                  sem_i0, sem_i1, sem_i2, sem_i3):
        c = lax.axis_index("c")
        s = lax.axis_index("s")
        w = s * NC + c

        pltpu.sync_copy(ones_hbm, ones_v)
        sem_i = [sem_i0, sem_i1, sem_i2, sem_i3]
        rows = [rows_a, rows_b]
        sem_r = [sem_a, sem_b]
        e0 = w * K

        def fetch_idx(chunk, row, sem):
            pltpu.async_copy(e_hbm.at[0, chunk], src_v.at[row], sem)
            pltpu.async_copy(e_hbm.at[1, chunk], dst_v.at[row], sem)

        def wait_idx(row, sem):
            pltpu.make_async_copy(e_hbm.at[0, 0], src_v.at[row], sem).wait()
            pltpu.make_async_copy(e_hbm.at[1, 0], dst_v.at[row], sem).wait()

        for t in range(4):
            fetch_idx(e0 + t, t, sem_i[t])
        wait_idx(0, sem_i[0])
        pltpu.async_copy(x_hbm.at[src_v.at[0]], rows_a, sem_a)
        def zrow(r, carry):
            for k2 in range(D // 16):
                rows_b[r, pl.ds(16 * k2, 16)] = jnp.zeros((16,), jnp.float32)
            return carry

        lax.fori_loop(0, B, zrow, 0)
        for q in range(RPT // B):
            pltpu.async_copy(rows_b,
                             acc_sh.at[pl.ds(s * RPT + q * B, B)], sem_d)
        pltpu.async_copy(zd_hbm.at[pl.ds(s * DPT, DPT)],
                         deg_sh.at[pl.ds(s * DPT, DPT)], sem_b)
        for q in range(RPT // B):
            pltpu.make_async_copy(rows_b,
                                  acc_sh.at[pl.ds(s * RPT + q * B, B)],
                                  sem_d).wait()
        pltpu.make_async_copy(zd_hbm.at[pl.ds(s * DPT, DPT)],
                              deg_sh.at[pl.ds(s * DPT, DPT)], sem_b).wait()
        wait_idx(1, sem_i[1])
        pltpu.async_copy(x_hbm.at[src_v.at[1]], rows_b, sem_b)
        plsc.subcore_barrier()

        def body(i, carry):
            j0 = 4 * i
            for t in range(4):
                j = j0 + t
                buf = rows[t % 2]
                sem = sem_r[t % 2]
                pltpu.make_async_copy(x_hbm.at[src_v.at[t]], buf, sem).wait()
                pltpu.async_copy(ones_v, deg_sh.at[dst_v.at[t]], sem_d,
                                 add=True)
                pltpu.sync_copy(buf, acc_sh.at[dst_v.at[t]], add=True)
                pltpu.make_async_copy(ones_v, deg_sh.at[dst_v.at[t]],
                                      sem_d).wait()
                @pl.when(j + 4 < K)
                def _():
                    fetch_idx(e0 + j + 4, t, sem_i[t])

                @pl.when(j + 2 < K)
                def _():
                    wait_idx((t + 2) % 4, sem_i[(t + 2) % 4])
                    pltpu.async_copy(x_hbm.at[src_v.at[(t + 2) % 4]],
                                     buf, sem)
            return carry

        lax.fori_loop(0, K // 4, body, 0)
        plsc.subcore_barrier()

        @pl.when(c == 0)
        def _():
            pltpu.async_copy(acc_sh.at[pl.ds(s * RPT, RPT)],
                             acc0_out.at[pl.ds(s * RPT, RPT)], sem_a)
            pltpu.async_copy(deg_sh.at[pl.ds(s * DPT, DPT)],
                             deg0_out.at[pl.ds(s * DPT, DPT)], sem_b)
            pltpu.make_async_copy(acc_sh.at[pl.ds(s * RPT, RPT)],
                                  acc0_out.at[pl.ds(s * RPT, RPT)],
                                  sem_a).wait()
            pltpu.make_async_copy(deg_sh.at[pl.ds(s * DPT, DPT)],
                                  deg0_out.at[pl.ds(s * DPT, DPT)],
                                  sem_b).wait()

        @pl.when(c == 1)
        def _():
            pltpu.async_copy(acc_sh.at[pl.ds(s * RPT, RPT)],
                             acc1_out.at[pl.ds(s * RPT, RPT)], sem_a)
            pltpu.async_copy(deg_sh.at[pl.ds(s * DPT, DPT)],
                             deg1_out.at[pl.ds(s * DPT, DPT)], sem_b)
            pltpu.make_async_copy(acc_sh.at[pl.ds(s * RPT, RPT)],
                                  acc1_out.at[pl.ds(s * RPT, RPT)],
                                  sem_a).wait()
            pltpu.make_async_copy(deg_sh.at[pl.ds(s * DPT, DPT)],
                                  deg1_out.at[pl.ds(s * DPT, DPT)],
                                  sem_b).wait()

    return sc_kernel(x, edge3, zeros_deg, ones_col)


def _finalize(a0, a1, d0, d1, x, W_msg, W_self, b2):
    BN = 2000
    grid = (N // BN,)

    def tc_kernel(a0_ref, a1_ref, d0_ref, d1_ref, x_ref, wm_ref, ws_ref,
                  b_ref, o_ref):
        deg = d0_ref[:, 0:1] + d1_ref[:, 0:1]
        agg = (a0_ref[...] + a1_ref[...]) * (1.0 / jnp.maximum(deg, 1.0))
        out = jnp.dot(agg, wm_ref[...], preferred_element_type=jnp.float32)
        out = out + jnp.dot(x_ref[...], ws_ref[...],
                            preferred_element_type=jnp.float32)
        o_ref[...] = jnp.maximum(out + b_ref[...], 0.0)

    return pl.pallas_call(
        tc_kernel,
        grid=grid,
        in_specs=[
            pl.BlockSpec((BN, D), lambda i: (i, 0)),
            pl.BlockSpec((BN, D), lambda i: (i, 0)),
            pl.BlockSpec((BN, DW), lambda i: (i, 0)),
            pl.BlockSpec((BN, DW), lambda i: (i, 0)),
            pl.BlockSpec((BN, D), lambda i: (i, 0)),
            pl.BlockSpec((D, D), lambda i: (0, 0)),
            pl.BlockSpec((D, D), lambda i: (0, 0)),
            pl.BlockSpec((1, D), lambda i: (0, 0)),
        ],
        out_specs=pl.BlockSpec((BN, D), lambda i: (i, 0)),
        out_shape=jax.ShapeDtypeStruct((N, D), jnp.float32),
    )(a0, a1, d0, d1, x, W_msg, W_self, b2)


def kernel(x, edge_index, W_msg, W_self, b):
    edge3 = edge_index.reshape(2, NW * K, B)
    zeros_deg = jnp.zeros((NDEG, DW), jnp.float32)
    ones_col = jnp.ones((B, DW), jnp.float32)
    a0, a1, dg0, dg1 = _sc_aggregate(x, edge3, zeros_deg, ones_col)
    return _finalize(a0, a1, dg0, dg1, x, W_msg, W_self, b.reshape(1, D))


# speedup vs baseline: 1.1294x; 1.1294x over previous
import functools

import jax
import jax.numpy as jnp
from jax import lax
from jax.experimental import pallas as pl
from jax.experimental.pallas import tpu as pltpu
from jax.experimental.pallas import tpu_sc as plsc

N = 10000
E = 320000
D = 128

NC = 2
NS = 16
NW = NC * NS
B = 125
K = E // (NW * B)
RPT = N // NS
DPT = 640
NDEG = NS * DPT
DW = 16


def _sc_aggregate(x, edge3, zeros_deg, ones_col):
    mesh = plsc.VectorSubcoreMesh(core_axis_name="c", subcore_axis_name="s")

    @functools.partial(
        pl.kernel,
        mesh=mesh,
        out_type=(
            jax.ShapeDtypeStruct((N, D), jnp.bfloat16),
            jax.ShapeDtypeStruct((N, D), jnp.bfloat16),
            jax.ShapeDtypeStruct((NDEG, DW), jnp.float32),
            jax.ShapeDtypeStruct((NDEG, DW), jnp.float32),
        ),
        scratch_types=[
            pltpu.VMEM((4, B), jnp.int32),
            pltpu.VMEM((4, B), jnp.int32),
            pltpu.VMEM((B, D), jnp.bfloat16),
            pltpu.VMEM((B, D), jnp.bfloat16),
            pltpu.VMEM((B, DW), jnp.float32),
            pltpu.VMEM_SHARED((N, D), jnp.bfloat16),
            pltpu.VMEM_SHARED((NDEG, DW), jnp.float32),
            pltpu.SemaphoreType.DMA,
            pltpu.SemaphoreType.DMA,
            pltpu.SemaphoreType.DMA,
            pltpu.SemaphoreType.DMA,
            pltpu.SemaphoreType.DMA,
            pltpu.SemaphoreType.DMA,
            pltpu.SemaphoreType.DMA,
        ],
        compiler_params=pltpu.CompilerParams(use_tc_tiling_on_sc=False),
    )
    def sc_kernel(x_hbm, e_hbm, zd_hbm, ones_hbm,
                  acc0_out, acc1_out, deg0_out, deg1_out,
                  src_v, dst_v, rows_a, rows_b, ones_v,
                  acc_sh, deg_sh, sem_a, sem_b, sem_d,
                  sem_i0, sem_i1, sem_i2, sem_i3):
        c = lax.axis_index("c")
        s = lax.axis_index("s")
        w = s * NC + c

        pltpu.sync_copy(ones_hbm, ones_v)
        sem_i = [sem_i0, sem_i1, sem_i2, sem_i3]
        rows = [rows_a, rows_b]
        sem_r = [sem_a, sem_b]
        e0 = w * K

        def fetch_idx(chunk, row, sem):
            pltpu.async_copy(e_hbm.at[0, chunk], src_v.at[row], sem)
            pltpu.async_copy(e_hbm.at[1, chunk], dst_v.at[row], sem)

        def wait_idx(row, sem):
            pltpu.make_async_copy(e_hbm.at[0, 0], src_v.at[row], sem).wait()
            pltpu.make_async_copy(e_hbm.at[1, 0], dst_v.at[row], sem).wait()

        for t in range(4):
            fetch_idx(e0 + t, t, sem_i[t])
        wait_idx(0, sem_i[0])
        pltpu.async_copy(x_hbm.at[src_v.at[0]], rows_a, sem_a)
        def zrow(r, carry):
            for k2 in range(D // 32):
                rows_b[r, pl.ds(32 * k2, 32)] = jnp.zeros((32,),
                                                          jnp.bfloat16)
            return carry

        lax.fori_loop(0, B, zrow, 0)
        for q in range(RPT // B):
            pltpu.async_copy(rows_b,
                             acc_sh.at[pl.ds(s * RPT + q * B, B)], sem_d)
        pltpu.async_copy(zd_hbm.at[pl.ds(s * DPT, DPT)],
                         deg_sh.at[pl.ds(s * DPT, DPT)], sem_b)
        for q in range(RPT // B):
            pltpu.make_async_copy(rows_b,
                                  acc_sh.at[pl.ds(s * RPT + q * B, B)],
                                  sem_d).wait()
        pltpu.make_async_copy(zd_hbm.at[pl.ds(s * DPT, DPT)],
                              deg_sh.at[pl.ds(s * DPT, DPT)], sem_b).wait()
        wait_idx(1, sem_i[1])
        pltpu.async_copy(x_hbm.at[src_v.at[1]], rows_b, sem_b)
        plsc.subcore_barrier()

        def body(i, carry):
            j0 = 4 * i
            for t in range(4):
                j = j0 + t
                buf = rows[t % 2]
                sem = sem_r[t % 2]
                pltpu.make_async_copy(x_hbm.at[src_v.at[t]], buf, sem).wait()
                pltpu.async_copy(ones_v, deg_sh.at[dst_v.at[t]], sem_d,
                                 add=True)
                pltpu.sync_copy(buf, acc_sh.at[dst_v.at[t]], add=True)
                pltpu.make_async_copy(ones_v, deg_sh.at[dst_v.at[t]],
                                      sem_d).wait()
                @pl.when(j + 4 < K)
                def _():
                    fetch_idx(e0 + j + 4, t, sem_i[t])

                @pl.when(j + 2 < K)
                def _():
                    wait_idx((t + 2) % 4, sem_i[(t + 2) % 4])
                    pltpu.async_copy(x_hbm.at[src_v.at[(t + 2) % 4]],
                                     buf, sem)
            return carry

        lax.fori_loop(0, K // 4, body, 0)
        plsc.subcore_barrier()

        @pl.when(c == 0)
        def _():
            pltpu.async_copy(acc_sh.at[pl.ds(s * RPT, RPT)],
                             acc0_out.at[pl.ds(s * RPT, RPT)], sem_a)
            pltpu.async_copy(deg_sh.at[pl.ds(s * DPT, DPT)],
                             deg0_out.at[pl.ds(s * DPT, DPT)], sem_b)
            pltpu.make_async_copy(acc_sh.at[pl.ds(s * RPT, RPT)],
                                  acc0_out.at[pl.ds(s * RPT, RPT)],
                                  sem_a).wait()
            pltpu.make_async_copy(deg_sh.at[pl.ds(s * DPT, DPT)],
                                  deg0_out.at[pl.ds(s * DPT, DPT)],
                                  sem_b).wait()

        @pl.when(c == 1)
        def _():
            pltpu.async_copy(acc_sh.at[pl.ds(s * RPT, RPT)],
                             acc1_out.at[pl.ds(s * RPT, RPT)], sem_a)
            pltpu.async_copy(deg_sh.at[pl.ds(s * DPT, DPT)],
                             deg1_out.at[pl.ds(s * DPT, DPT)], sem_b)
            pltpu.make_async_copy(acc_sh.at[pl.ds(s * RPT, RPT)],
                                  acc1_out.at[pl.ds(s * RPT, RPT)],
                                  sem_a).wait()
            pltpu.make_async_copy(deg_sh.at[pl.ds(s * DPT, DPT)],
                                  deg1_out.at[pl.ds(s * DPT, DPT)],
                                  sem_b).wait()

    return sc_kernel(x, edge3, zeros_deg, ones_col)


def _finalize(a0, a1, d0, d1, x, W_msg, W_self, b2):
    BN = 2000
    grid = (N // BN,)

    def tc_kernel(a0_ref, a1_ref, d0_ref, d1_ref, x_ref, wm_ref, ws_ref,
                  b_ref, o_ref):
        deg = d0_ref[:, 0:1] + d1_ref[:, 0:1]
        asum = (a0_ref[...].astype(jnp.float32) +
                a1_ref[...].astype(jnp.float32))
        agg = asum * (1.0 / jnp.maximum(deg, 1.0))
        out = jnp.dot(agg, wm_ref[...], preferred_element_type=jnp.float32)
        out = out + jnp.dot(x_ref[...], ws_ref[...],
                            preferred_element_type=jnp.float32)
        o_ref[...] = jnp.maximum(out + b_ref[...], 0.0)

    return pl.pallas_call(
        tc_kernel,
        grid=grid,
        in_specs=[
            pl.BlockSpec((BN, D), lambda i: (i, 0)),
            pl.BlockSpec((BN, D), lambda i: (i, 0)),
            pl.BlockSpec((BN, DW), lambda i: (i, 0)),
            pl.BlockSpec((BN, DW), lambda i: (i, 0)),
            pl.BlockSpec((BN, D), lambda i: (i, 0)),
            pl.BlockSpec((D, D), lambda i: (0, 0)),
            pl.BlockSpec((D, D), lambda i: (0, 0)),
            pl.BlockSpec((1, D), lambda i: (0, 0)),
        ],
        out_specs=pl.BlockSpec((BN, D), lambda i: (i, 0)),
        out_shape=jax.ShapeDtypeStruct((N, D), jnp.float32),
    )(a0, a1, d0, d1, x, W_msg, W_self, b2)


def kernel(x, edge_index, W_msg, W_self, b):
    xb = x.astype(jnp.bfloat16)
    edge3 = edge_index.reshape(2, NW * K, B)
    zeros_deg = jnp.zeros((NDEG, DW), jnp.float32)
    ones_col = jnp.ones((B, DW), jnp.float32)
    a0, a1, dg0, dg1 = _sc_aggregate(xb, edge3, zeros_deg, ones_col)
    return _finalize(a0, a1, dg0, dg1, x, W_msg, W_self, b.reshape(1, D))
